# speedup vs baseline: 1.1620x; 1.1620x over previous
import functools

import jax
import jax.numpy as jnp
from jax import lax
from jax.experimental import pallas as pl
from jax.experimental.pallas import tpu as pltpu
from jax.experimental.pallas import tpu_sc as plsc

_B = 4096 * 26
_D = 128
_NC = 2
_NS = 16
_NW = _NC * _NS
_BPW = _B // _NW
_CH = 128
_NCHUNK = _BPW // _CH


def _make_gather():
  mesh = plsc.VectorSubcoreMesh(core_axis_name="c", subcore_axis_name="s")

  @functools.partial(
      pl.kernel,
      mesh=mesh,
      out_type=jax.ShapeDtypeStruct((_B, _D), jnp.float32),
      scratch_types=[
          pltpu.VMEM((_NCHUNK, _CH), jnp.int32),
          pltpu.VMEM((_CH, _D), jnp.float32),
          pltpu.SemaphoreType.DMA,
      ],
  )
  def k(idx_hbm, table_hbm, out_hbm, idx_v, rows_v, sem):
    wid = lax.axis_index("s") * _NC + lax.axis_index("c")
    base = wid * _BPW
    pltpu.sync_copy(idx_hbm.at[wid], idx_v)

    def body(j, carry):
      pltpu.async_copy(table_hbm.at[idx_v.at[j]], rows_v, sem).wait()
      pltpu.sync_copy(rows_v, out_hbm.at[pl.ds(base + j * _CH, _CH)])
      return carry

    lax.fori_loop(0, _NCHUNK, body, 0)

  return k


_gather = _make_gather()


def kernel(x, weight):
  idx = x.reshape(_NW, _NCHUNK, _CH).astype(jnp.int32)
  out = _gather(idx, weight)
  return out.reshape(4096, 26, _D)


# speedup vs baseline: 1.2803x; 1.1018x over previous
import functools

import jax
import jax.numpy as jnp
from jax import lax
from jax.experimental import pallas as pl
from jax.experimental.pallas import tpu as pltpu
from jax.experimental.pallas import tpu_sc as plsc

_B = 4096 * 26
_D = 128
_NC = 2
_NS = 16
_NW = _NC * _NS
_BPW = _B // _NW
_CH = 104
_NCHUNK = _BPW // _CH
_NBUF = 4
_NGRP = _NCHUNK // _NBUF


def _make_gather():
  mesh = plsc.VectorSubcoreMesh(core_axis_name="c", subcore_axis_name="s")

  @functools.partial(
      pl.kernel,
      mesh=mesh,
      out_type=jax.ShapeDtypeStruct((_B, _D), jnp.float32),
      scratch_types=[
          pltpu.VMEM((_NCHUNK, _CH), jnp.int32),
          pltpu.VMEM((_NBUF, _CH, _D), jnp.float32),
      ] + [pltpu.SemaphoreType.DMA] * (2 * _NBUF),
  )
  def k(idx_hbm, table_hbm, out_hbm, idx_v, rows_v, *sems):
    gsem = sems[:_NBUF]
    osem = sems[_NBUF:]
    wid = lax.axis_index("s") * _NC + lax.axis_index("c")
    base = wid * _BPW
    pltpu.sync_copy(idx_hbm.at[wid], idx_v)

    def group(g, carry):
      for b in range(_NBUF):
        j = g * _NBUF + b

        @pl.when(g > 0)
        def _(b=b):
          pltpu.make_async_copy(
              rows_v.at[b], out_hbm.at[pl.ds(base, _CH)], osem[b]).wait()

        pltpu.make_async_copy(
            table_hbm.at[idx_v.at[j]], rows_v.at[b], gsem[b]).start()
      for b in range(_NBUF):
        j = g * _NBUF + b
        pltpu.make_async_copy(
            table_hbm.at[idx_v.at[j]], rows_v.at[b], gsem[b]).wait()
        pltpu.make_async_copy(
            rows_v.at[b], out_hbm.at[pl.ds(base + j * _CH, _CH)],
            osem[b]).start()
      return carry

    lax.fori_loop(0, _NGRP, group, 0)
    for b in range(_NBUF):
      pltpu.make_async_copy(
          rows_v.at[b], out_hbm.at[pl.ds(base, _CH)], osem[b]).wait()

  return k


_gather = _make_gather()


def kernel(x, weight):
  idx = x.reshape(_NW, _NCHUNK, _CH).astype(jnp.int32)
  out = _gather(idx, weight)
  return out.reshape(4096, 26, _D)


# speedup vs baseline: 1.9175x; 1.4977x over previous
import functools

import jax
import jax.numpy as jnp
from jax import lax
from jax.experimental import pallas as pl
from jax.experimental.pallas import tpu as pltpu
from jax.experimental.pallas import tpu_sc as plsc

_S = 26
_SP = 32
_NB = 4096
_B = _NB * _S
_D = 128
_NC = 2
_NS = 16
_NW = _NC * _NS
_XPW = _NB // _NW
_BPW = _B // _NW
_XCH = 4
_CH = _XCH * _S
_NCHUNK = _XPW // _XCH
_NBUF = 4
_NGRP = _NCHUNK // _NBUF


def _make_gather():
  mesh = plsc.VectorSubcoreMesh(core_axis_name="c", subcore_axis_name="s")

  @functools.partial(
      pl.kernel,
      mesh=mesh,
      compiler_params=pltpu.CompilerParams(use_tc_tiling_on_sc=False),
      out_type=jax.ShapeDtypeStruct((_NB * _SP, _D), jnp.float32),
      scratch_types=[
          pltpu.VMEM((_BPW,), jnp.int32),
          pltpu.VMEM((_NBUF, _CH, _D), jnp.float32),
      ] + [pltpu.SemaphoreType.DMA] * (2 * _NBUF),
  )
  def k(idx_hbm, table_hbm, out_hbm, idx_v, rows_v, *sems):
    gsem = sems[:_NBUF]
    osem = sems[_NBUF:]
    wid = lax.axis_index("s") * _NC + lax.axis_index("c")
    xbase = wid * _XPW
    pltpu.sync_copy(idx_hbm.at[wid], idx_v)

    def store_descs(b, j):
      xr = xbase + j * _XCH
      return [
          pltpu.make_async_copy(
              rows_v.at[b, pl.ds(s * _S, _S)],
              out_hbm.at[pl.ds((xr + s) * _SP, _S)],
              osem[b])
          for s in range(_XCH)
      ]

    def group(g, carry):
      for b in range(_NBUF):
        j = g * _NBUF + b

        @pl.when(g > 0)
        def _(b=b, j=j):
          for c in store_descs(b, j - _NBUF):
            c.wait()

        pltpu.make_async_copy(
            table_hbm.at[idx_v.at[pl.ds(j * _CH, _CH)]],
            rows_v.at[b], gsem[b]).start()
      for b in range(_NBUF):
        j = g * _NBUF + b
        pltpu.make_async_copy(
            table_hbm.at[idx_v.at[pl.ds(j * _CH, _CH)]],
            rows_v.at[b], gsem[b]).wait()
        for c in store_descs(b, j):
          c.start()
      return carry

    lax.fori_loop(0, _NGRP, group, 0)
    for b in range(_NBUF):
      for c in store_descs(b, _NCHUNK - _NBUF + b):
        c.wait()

  return k


_gather = _make_gather()


def kernel(x, weight):
  idx = x.reshape(_NW, _BPW).astype(jnp.int32)
  out = _gather(idx, weight)
  return out.reshape(_NB, _SP, _D)[:, :_S, :]


# speedup vs baseline: 2.0057x; 1.0460x over previous
import functools

import jax
import jax.numpy as jnp
from jax import lax
from jax.experimental import pallas as pl
from jax.experimental.pallas import tpu as pltpu
from jax.experimental.pallas import tpu_sc as plsc

_S = 26
_NB = 4096
_D = 128
_NC = 2
_NS = 16
_NW = _NC * _NS
_XPW = _NB // _NW
_XCH = 4
_NCHUNK = _XPW // _XCH
_NBUF = 4
_NGRP = _NCHUNK // _NBUF


def _make_gather():
  mesh = plsc.VectorSubcoreMesh(core_axis_name="c", subcore_axis_name="s")

  @functools.partial(
      pl.kernel,
      mesh=mesh,
      out_type=jax.ShapeDtypeStruct((_NB, _S, _D), jnp.float32),
      scratch_types=[
          pltpu.VMEM((_XPW, _S), jnp.int32),
          pltpu.VMEM((_NBUF, _XCH, _S, _D), jnp.float32),
      ] + [pltpu.SemaphoreType.DMA] * (2 * _NBUF),
  )
  def k(idx_hbm, table_hbm, out_hbm, idx_v, rows_v, *sems):
    gsem = sems[:_NBUF]
    osem = sems[_NBUF:]
    wid = lax.axis_index("s") * _NC + lax.axis_index("c")
    xbase = wid * _XPW
    pltpu.sync_copy(idx_hbm.at[pl.ds(xbase, _XPW)], idx_v)

    def gather_descs(b, j):
      return [
          pltpu.make_async_copy(
              table_hbm.at[idx_v.at[j * _XCH + s]],
              rows_v.at[b, s], gsem[b])
          for s in range(_XCH)
      ]

    def store_desc(b, j):
      return pltpu.make_async_copy(
          rows_v.at[b], out_hbm.at[pl.ds(xbase + j * _XCH, _XCH)], osem[b])

    def group(g, carry):
      for b in range(_NBUF):
        j = g * _NBUF + b

        @pl.when(g > 0)
        def _(b=b, j=j):
          store_desc(b, j - _NBUF).wait()

        for c in gather_descs(b, j):
          c.start()
      for b in range(_NBUF):
        j = g * _NBUF + b
        for c in gather_descs(b, j):
          c.wait()
        store_desc(b, j).start()
      return carry

    lax.fori_loop(0, _NGRP, group, 0)
    for b in range(_NBUF):
      store_desc(b, _NCHUNK - _NBUF + b).wait()

  return k


_gather = _make_gather()


def kernel(x, weight):
  return _gather(x.astype(jnp.int32), weight)


# speedup vs baseline: 3.5883x; 1.7891x over previous
import functools

import jax
import jax.numpy as jnp
from jax import lax
from jax.experimental import pallas as pl
from jax.experimental.pallas import tpu as pltpu
from jax.experimental.pallas import tpu_sc as plsc

_S = 26
_NB = 4096
_D = 128
_NC = 2
_NS = 16
_NW = _NC * _NS
_CPW = _NB // _NW
_CH = 64
_HB = _CPW // _CH
_NCHUNK = _S * _HB
_NBUF = 4
_NGRP = _NCHUNK // _NBUF


def _make_gather():
  mesh = plsc.VectorSubcoreMesh(core_axis_name="c", subcore_axis_name="s")

  @functools.partial(
      pl.kernel,
      mesh=mesh,
      out_type=jax.ShapeDtypeStruct((_S, _NB, _D), jnp.float32),
      scratch_types=[
          pltpu.VMEM((_S, _CPW), jnp.int32),
          pltpu.VMEM((_NBUF, _CH, _D), jnp.float32),
      ] + [pltpu.SemaphoreType.DMA] * (2 * _NBUF),
  )
  def k(idx_hbm, table_hbm, out_hbm, idx_v, rows_v, *sems):
    gsem = sems[:_NBUF]
    osem = sems[_NBUF:]
    wid = lax.axis_index("s") * _NC + lax.axis_index("c")
    cbase = wid * _CPW
    pltpu.sync_copy(idx_hbm.at[:, pl.ds(cbase, _CPW)], idx_v)

    def gather_desc(b, j):
      s = j // _HB
      h = j % _HB
      return pltpu.make_async_copy(
          table_hbm.at[idx_v.at[s, pl.ds(h * _CH, _CH)]],
          rows_v.at[b], gsem[b])

    def store_desc(b, j):
      s = j // _HB
      h = j % _HB
      return pltpu.make_async_copy(
          rows_v.at[b], out_hbm.at[s, pl.ds(cbase + h * _CH, _CH)], osem[b])

    def group(g, carry):
      for b in range(_NBUF):
        j = g * _NBUF + b

        @pl.when(g > 0)
        def _(b=b, j=j):
          store_desc(b, j - _NBUF).wait()

        gather_desc(b, j).start()
      for b in range(_NBUF):
        j = g * _NBUF + b
        gather_desc(b, j).wait()
        store_desc(b, j).start()
      return carry

    lax.fori_loop(0, _NGRP, group, 0)
    for b in range(_NBUF):
      store_desc(b, _NCHUNK - _NBUF + b).wait()

  return k


_gather = _make_gather()


def kernel(x, weight):
  out_t = _gather(x.T.astype(jnp.int32), weight)
  return out_t.transpose(1, 0, 2)


# speedup vs baseline: 3.7000x; 1.0311x over previous
import functools

import jax
import jax.numpy as jnp
from jax import lax
from jax.experimental import pallas as pl
from jax.experimental.pallas import tpu as pltpu
from jax.experimental.pallas import tpu_sc as plsc

_S = 26
_NB = 4096
_D = 128
_NC = 2
_NS = 16
_NW = _NC * _NS
_CPW = _NB // _NW
_CH = 64
_HB = _CPW // _CH
_NCHUNK = _S * _HB
_NBUF = 13
_NGRP = _NCHUNK // _NBUF


def _make_gather():
  mesh = plsc.VectorSubcoreMesh(core_axis_name="c", subcore_axis_name="s")

  @functools.partial(
      pl.kernel,
      mesh=mesh,
      out_type=jax.ShapeDtypeStruct((_S, _NB, _D), jnp.float32),
      scratch_types=[
          pltpu.VMEM((_S, _CPW), jnp.int32),
          pltpu.VMEM((_NBUF, _CH, _D), jnp.float32),
      ] + [pltpu.SemaphoreType.DMA] * (2 * _NBUF),
  )
  def k(idx_hbm, table_hbm, out_hbm, idx_v, rows_v, *sems):
    gsem = sems[:_NBUF]
    osem = sems[_NBUF:]
    wid = lax.axis_index("s") * _NC + lax.axis_index("c")
    cbase = wid * _CPW
    pltpu.sync_copy(idx_hbm.at[:, pl.ds(cbase, _CPW)], idx_v)

    def gather_desc(b, j):
      s = j // _HB
      h = j % _HB
      return pltpu.make_async_copy(
          table_hbm.at[idx_v.at[s, pl.ds(h * _CH, _CH)]],
          rows_v.at[b], gsem[b])

    def store_desc(b, j):
      s = j // _HB
      h = j % _HB
      return pltpu.make_async_copy(
          rows_v.at[b], out_hbm.at[s, pl.ds(cbase + h * _CH, _CH)], osem[b])

    def group(g, carry):
      for b in range(_NBUF):
        j = g * _NBUF + b

        @pl.when(g > 0)
        def _(b=b, j=j):
          store_desc(b, j - _NBUF).wait()

        gather_desc(b, j).start()
      for b in range(_NBUF):
        j = g * _NBUF + b
        gather_desc(b, j).wait()
        store_desc(b, j).start()
      return carry

    lax.fori_loop(0, _NGRP, group, 0)
    for b in range(_NBUF):
      store_desc(b, _NCHUNK - _NBUF + b).wait()

  return k


_gather = _make_gather()


def kernel(x, weight):
  out_t = _gather(x.T.astype(jnp.int32), weight)
  return out_t.transpose(1, 0, 2)
